# baseline (speedup 1.0000x reference)
import functools

import jax
import jax.numpy as jnp
from jax import lax
from jax.experimental import pallas as pl
from jax.experimental.pallas import tpu as pltpu
from jax.experimental.pallas import tpu_sc as plsc

_B = 4096
_D = 128
_V = 100000


def _sc_gather(target_word, emb_table):
    info = plsc.get_sparse_core_info()
    nc, ns = info.num_cores, info.num_subcores
    nw = nc * ns
    b_per_w = _B // nw
    mesh = plsc.VectorSubcoreMesh(core_axis_name="c", subcore_axis_name="s")

    @functools.partial(
        pl.kernel,
        mesh=mesh,
        out_type=jax.ShapeDtypeStruct((_B, _D), jnp.float32),
        scratch_types=[
            pltpu.VMEM((b_per_w,), jnp.int32),
            pltpu.VMEM((b_per_w, _D), jnp.float32),
            pltpu.SemaphoreType.DMA,
        ],
    )
    def gather_kernel(idx_hbm, table_hbm, out_hbm, idx_v, rows_v, sem):
        wid = lax.axis_index("s") * nc + lax.axis_index("c")
        base = wid * b_per_w
        pltpu.sync_copy(idx_hbm.at[pl.ds(base, b_per_w)], idx_v)
        pltpu.async_copy(table_hbm.at[idx_v], rows_v, sem).wait()
        pltpu.sync_copy(rows_v, out_hbm.at[pl.ds(base, b_per_w)])

    return gather_kernel(target_word, emb_table)


_BN = 1024


def _mm_kernel(emb_ref, w_ref, b_ref, out_ref):
    out_ref[...] = lax.dot_general(
        emb_ref[...], w_ref[...],
        dimension_numbers=(((1,), (1,)), ((), ())),
        preferred_element_type=jnp.float32,
    ) + b_ref[...]


def _tc_project(embedded, W, b2d):
    grid = (pl.cdiv(_V, _BN),)
    return pl.pallas_call(
        _mm_kernel,
        grid=grid,
        in_specs=[
            pl.BlockSpec((_B, _D), lambda j: (0, 0)),
            pl.BlockSpec((_BN, _D), lambda j: (j, 0)),
            pl.BlockSpec((1, _BN), lambda j: (0, j)),
        ],
        out_specs=pl.BlockSpec((_B, _BN), lambda j: (0, j)),
        out_shape=jax.ShapeDtypeStruct((_B, _V), jnp.float32),
    )(embedded, W, b2d)


def kernel(target_word, emb_table, W, b):
    embedded = _sc_gather(target_word.astype(jnp.int32), emb_table)
    return _tc_project(embedded, W, b.reshape(1, _V))


# speedup vs baseline: 1.0113x; 1.0113x over previous
import functools

import jax
import jax.numpy as jnp
from jax import lax
from jax.experimental import pallas as pl
from jax.experimental.pallas import tpu as pltpu
from jax.experimental.pallas import tpu_sc as plsc

_B = 4096
_D = 128
_V = 100000


def _sc_gather(target_word, emb_table):
    info = plsc.get_sparse_core_info()
    nc, ns = info.num_cores, info.num_subcores
    nw = nc * ns
    b_per_w = _B // nw
    mesh = plsc.VectorSubcoreMesh(core_axis_name="c", subcore_axis_name="s")

    @functools.partial(
        pl.kernel,
        mesh=mesh,
        out_type=jax.ShapeDtypeStruct((_B, _D), jnp.float32),
        scratch_types=[
            pltpu.VMEM((b_per_w,), jnp.int32),
            pltpu.VMEM((b_per_w, _D), jnp.float32),
            pltpu.SemaphoreType.DMA,
        ],
    )
    def gather_kernel(idx_hbm, table_hbm, out_hbm, idx_v, rows_v, sem):
        wid = lax.axis_index("s") * nc + lax.axis_index("c")
        base = wid * b_per_w
        pltpu.sync_copy(idx_hbm.at[pl.ds(base, b_per_w)], idx_v)
        pltpu.async_copy(table_hbm.at[idx_v], rows_v, sem).wait()
        pltpu.sync_copy(rows_v, out_hbm.at[pl.ds(base, b_per_w)])

    return gather_kernel(target_word, emb_table)


_BN = 512
_NSLOT = 5
_NT = (_V + _BN - 1) // _BN
_TAIL = _V - (_NT - 1) * _BN


def _mm_kernel(emb_ref, w_ref, b_ref, out_hbm, acc, acc_tail, sems, sem_tail):
    j = pl.program_id(0)
    slot = lax.rem(j, _NSLOT)

    res = lax.dot_general(
        emb_ref[...], w_ref[...],
        dimension_numbers=(((1,), (1,)), ((), ())),
        preferred_element_type=jnp.float32,
    ) + b_ref[...]

    @pl.when(jnp.logical_and(j >= _NSLOT, j < _NT - 1))
    def _wait_prev():
        pltpu.make_async_copy(
            acc.at[slot],
            out_hbm.at[:, pl.ds(0, _BN)],
            sems.at[slot],
        ).wait()

    @pl.when(j < _NT - 1)
    def _start_full():
        acc[slot] = res
        pltpu.make_async_copy(
            acc.at[slot],
            out_hbm.at[:, pl.ds(j * _BN, _BN)],
            sems.at[slot],
        ).start()

    @pl.when(j == _NT - 1)
    def _start_tail_and_drain():
        acc_tail[...] = res[:, :_TAIL]
        pltpu.make_async_copy(
            acc_tail,
            out_hbm.at[:, pl.ds((_NT - 1) * _BN, _TAIL)],
            sem_tail,
        ).start()
        for s in range(_NSLOT):
            pltpu.make_async_copy(
                acc.at[s],
                out_hbm.at[:, pl.ds(0, _BN)],
                sems.at[s],
            ).wait()
        pltpu.make_async_copy(
            acc_tail,
            out_hbm.at[:, pl.ds((_NT - 1) * _BN, _TAIL)],
            sem_tail,
        ).wait()


def _tc_project(embedded, W, b2d):
    return pl.pallas_call(
        _mm_kernel,
        grid=(_NT,),
        in_specs=[
            pl.BlockSpec((_B, _D), lambda j: (0, 0)),
            pl.BlockSpec((_BN, _D), lambda j: (j, 0)),
            pl.BlockSpec((1, _BN), lambda j: (0, j)),
        ],
        out_specs=pl.BlockSpec(memory_space=pl.ANY),
        out_shape=jax.ShapeDtypeStruct((_B, _V), jnp.float32),
        scratch_shapes=[
            pltpu.VMEM((_NSLOT, _B, _BN), jnp.float32),
            pltpu.VMEM((_B, _TAIL), jnp.float32),
            pltpu.SemaphoreType.DMA((_NSLOT,)),
            pltpu.SemaphoreType.DMA,
        ],
    )(embedded, W, b2d)


def kernel(target_word, emb_table, W, b):
    embedded = _sc_gather(target_word.astype(jnp.int32), emb_table)
    return _tc_project(embedded, W, b.reshape(1, _V))
